# Initial kernel scaffold; baseline (speedup 1.0000x reference)
import jax
import jax.numpy as jnp
from jax.experimental import pallas as pl


def kernel(expert_out_0, expert_out_1, expert_out_2, expert_out_3, expert_out_4, expert_out_5, expert_out_6, expert_out_7, shared_hidden, router_w):
    raise NotImplementedError("write your pallas kernel here")


# speedup vs baseline: 2.6795x; 2.6795x over previous
import functools

import jax
import jax.numpy as jnp
from jax.experimental import pallas as pl

_N_EXPERTS = 8
_TOKENS = 2 * 2048
_D = 1024
_TB = 256


def _body(sh_ref, e0, e1, e2, e3, e4, e5, e6, e7, rw_ref, fused_ref, w_ref):
    sh = sh_ref[...]
    rw = rw_ref[...]
    logits = jax.lax.dot_general(
        sh, rw, dimension_numbers=(((1,), (1,)), ((), ())),
        preferred_element_type=jnp.float32)

    eids = jax.lax.broadcasted_iota(jnp.int32, logits.shape, 1)
    m0 = jnp.max(logits, axis=1, keepdims=True)
    i0 = jnp.min(jnp.where(logits == m0, eids, _N_EXPERTS), axis=1,
                 keepdims=True)
    masked = jnp.where(eids == i0, -jnp.inf, logits)
    m1 = jnp.max(masked, axis=1, keepdims=True)
    i1 = jnp.min(jnp.where(masked == m1, eids, _N_EXPERTS), axis=1,
                 keepdims=True)

    t = jnp.exp(m1 - m0)
    p0 = 1.0 / (1.0 + t)
    p1 = t * p0

    w = jnp.where(eids == i0, p0, 0.0) + jnp.where(eids == i1, p1, 0.0)
    w_ref[...] = w

    experts = (e0, e1, e2, e3, e4, e5, e6, e7)
    acc = w[:, 0:1] * experts[0][...]
    for e in range(1, _N_EXPERTS):
        acc = acc + w[:, e:e + 1] * experts[e][...]
    fused_ref[...] = acc


@functools.partial(jax.jit, static_argnums=())
def _fused(shared2d, experts2d, router_w):
    grid = (_TOKENS // _TB,)
    tok_spec = pl.BlockSpec((_TB, _D), lambda i: (i, 0))
    fused, weights = pl.pallas_call(
        _body,
        grid=grid,
        in_specs=[tok_spec] + [tok_spec] * _N_EXPERTS
        + [pl.BlockSpec((_N_EXPERTS, _D), lambda i: (0, 0))],
        out_specs=[tok_spec, pl.BlockSpec((_TB, _N_EXPERTS), lambda i: (i, 0))],
        out_shape=[
            jax.ShapeDtypeStruct((_TOKENS, _D), jnp.float32),
            jax.ShapeDtypeStruct((_TOKENS, _N_EXPERTS), jnp.float32),
        ],
    )(shared2d, *experts2d, router_w)
    return fused, weights


def kernel(expert_out_0, expert_out_1, expert_out_2, expert_out_3,
           expert_out_4, expert_out_5, expert_out_6, expert_out_7,
           shared_hidden, router_w):
    B, T, D = shared_hidden.shape
    experts2d = [e.reshape(B * T, D) for e in
                 (expert_out_0, expert_out_1, expert_out_2, expert_out_3,
                  expert_out_4, expert_out_5, expert_out_6, expert_out_7)]
    fused, weights = _fused(shared_hidden.reshape(B * T, D), experts2d,
                            router_w)
    return (fused.reshape(B, T, D), weights.reshape(B, T, _N_EXPERTS))
